# baseline (speedup 1.0000x reference)
import functools
import jax
import jax.numpy as jnp
from jax import lax
from jax.experimental import pallas as pl
from jax.experimental.pallas import tpu as pltpu, tpu_sc as plsc

NC = 2
NS = 16
NT = NC * NS
CH = 128
CH3 = 96
FRAC0 = 0.587
L = 16


def _k1_body(x_ref, w_ref, p_ref, h_ref, a1_ref, a2_ref):
    h = jnp.dot(x_ref[...], w_ref[...], preferred_element_type=jnp.float32)
    h = h + p_ref[0][None, :]
    h_ref[...] = h
    a1_ref[...] = jnp.sum(h * p_ref[1][None, :], axis=1) + p_ref[3, 0]
    a2_ref[...] = jnp.sum(h * p_ref[2][None, :], axis=1)


def _k1(x, W, params, n, d):
    return pl.pallas_call(
        _k1_body,
        out_shape=[
            jax.ShapeDtypeStruct((n, d), jnp.float32),
            jax.ShapeDtypeStruct((n,), jnp.float32),
            jax.ShapeDtypeStruct((n,), jnp.float32),
        ],
    )(x, W, params)


def _k2_body(n, ept0, ept1, src_hbm, dst_hbm, a1_hbm, a2_hbm,
             t_out, exu_out, s_v, d_v, idx2, val2, a1g_b, a2g_b, exu_v,
             sem, sem2):
    eptm = max(ept0, ept1)
    cid = lax.axis_index("c")
    sid = lax.axis_index("s")
    is0 = cid == 0
    base = jnp.where(is0, sid * ept0, NS * ept0 + sid * ept1)
    qc = jnp.where(is0, ept0 // CH, ept1 // CH)
    pltpu.sync_copy(src_hbm.at[pl.ds(base, eptm)], s_v)
    pltpu.sync_copy(dst_hbm.at[pl.ds(base, eptm)], d_v)

    def build(j, _):
        for c in range(CH // L):
            off = j * CH + c * L
            sv = s_v[pl.ds(off, L)]
            dv = d_v[pl.ds(off, L)]
            idx2[j, pl.ds(c * L, L)] = sv * n + dv
            val2[j, pl.ds(c * L, L)] = (
                base + off + lax.iota(jnp.int32, L))
        return 0

    lax.fori_loop(0, qc, build, 0)

    def scat(j, _):
        dsc = pltpu.async_copy(val2.at[j], t_out.at[idx2.at[j]], sem)
        g2 = pltpu.async_copy(
            a1_hbm.at[s_v.at[pl.ds(j * CH, CH)]], a1g_b, sem2)
        g3 = pltpu.async_copy(
            a2_hbm.at[d_v.at[pl.ds(j * CH, CH)]], a2g_b, sem2)
        g2.wait(); g3.wait()
        for c in range(CH // L):
            e = a1g_b[pl.ds(c * L, L)] + a2g_b[pl.ds(c * L, L)]
            e = jnp.where(e > 0, e, e * jnp.float32(0.01))
            exu_v[pl.ds(j * CH + c * L, L)] = jnp.exp(e)
        dsc.wait()
        return 0

    lax.fori_loop(0, qc, scat, 0)

    @pl.when(is0)
    def _():
        pltpu.sync_copy(exu_v.at[pl.ds(0, ept0)],
                        exu_out.at[pl.ds(base, ept0)])

    @pl.when(jnp.logical_not(is0))
    def _():
        pltpu.sync_copy(exu_v.at[pl.ds(0, ept1)],
                        exu_out.at[pl.ds(base, ept1)])


def _k2(src_full, dst_full, a1, a2, n, ept0, ept1, totp):
    nn = n * n
    eptm = max(ept0, ept1)
    body = functools.partial(_k2_body, n, ept0, ept1)
    return pl.kernel(
        body,
        out_type=[
            jax.ShapeDtypeStruct((nn,), jnp.int32),
            jax.ShapeDtypeStruct((totp,), jnp.float32),
        ],
        mesh=plsc.VectorSubcoreMesh(
            core_axis_name="c", subcore_axis_name="s",
            num_cores=NC, num_subcores=NS),
        compiler_params=pltpu.CompilerParams(needs_layout_passes=False),
        scratch_types=[
            pltpu.VMEM((eptm,), jnp.int32),
            pltpu.VMEM((eptm,), jnp.int32),
            pltpu.VMEM((eptm // CH, CH), jnp.int32),
            pltpu.VMEM((eptm // CH, CH), jnp.int32),
            pltpu.VMEM((CH,), jnp.float32),
            pltpu.VMEM((CH,), jnp.float32),
            pltpu.VMEM((eptm,), jnp.float32),
            pltpu.SemaphoreType.DMA,
            pltpu.SemaphoreType.DMA,
        ],
    )(src_full, dst_full, a1, a2)


def _k3_body(n, npad, d, ept0, ept1,
             src_hbm, dst_hbm, t_hbm, h_hbm, exu_hbm,
             den_out, num_out,
             d_v, exu_sv, src2, eid_b, rows, w_v, ex_v,
             num_sh, den_sh, sem, sem2, sem3):
    ephm = max(ept0, ept1) // 2
    cid = lax.axis_index("c")
    sid = lax.axis_index("s")
    is0 = cid == 0
    base = jnp.where(is0, sid * ept0, NS * ept0 + sid * ept1)
    qh = jnp.where(is0, ept0 // (2 * CH3), ept1 // (2 * CH3))
    stripe = npad // NS

    zero16 = jnp.zeros((L,), jnp.float32)

    def zrows(i, _):
        for c in range(d // L):
            rows[i, pl.ds(c * L, L)] = zero16
        return 0
    lax.fori_loop(0, CH3, zrows, 0)

    zoffs = sorted({min(t * CH3, stripe - CH3)
                    for t in range(-(-stripe // CH3))})
    for t in zoffs:
        pltpu.sync_copy(rows, num_sh.at[pl.ds(sid * stripe + t, CH3)])
    doffs = sorted({min(t * d, stripe - d) for t in range(-(-stripe // d))})
    for t in doffs:
        pltpu.sync_copy(rows.at[0],
                        den_sh.at[pl.ds(sid * stripe + t, d)])
    plsc.subcore_barrier()

    def half_loop(half):
        hoff = half * qh * CH3
        pltpu.sync_copy(dst_hbm.at[pl.ds(base + hoff, ephm)], d_v)
        pltpu.sync_copy(exu_hbm.at[pl.ds(base + hoff, ephm)], exu_sv)

        def chunk(j, _):
            pltpu.sync_copy(
                src_hbm.at[pl.ds(base + hoff + j * CH3, CH3)], src2.at[j])
            for c in range(CH3 // L):
                off = j * CH3 + c * L
                eid_b[pl.ds(c * L, L)] = (
                    src2[j, pl.ds(c * L, L)] * n + d_v[pl.ds(off, L)])
            g1 = pltpu.async_copy(t_hbm.at[eid_b], w_v, sem)
            cp = pltpu.async_copy(
                h_hbm.at[d_v.at[pl.ds(j * CH3, CH3)]], rows, sem2)
            g1.wait()
            for c in range(CH3 // L):
                off = j * CH3 + c * L
                k = base + hoff + off + lax.iota(jnp.int32, L)
                ex = jnp.where(w_v[pl.ds(c * L, L)] == k,
                               exu_sv[pl.ds(off, L)], jnp.float32(0.0))
                ex_v[pl.ds(c * L, L)] = ex
            dn = pltpu.async_copy(ex_v, den_sh.at[src2.at[j]], sem3,
                                  add=True)
            cp.wait()

            def scale(g, _):
                exg = ex_v[pl.ds(g * L, L)]
                for lane in range(L):
                    i = g * L + lane
                    s = jnp.full((L,), exg[lane], jnp.float32)
                    for c in range(d // L):
                        rows[i, pl.ds(c * L, L)] = (
                            rows[i, pl.ds(c * L, L)] * s)
                return 0
            lax.fori_loop(0, CH3 // L, scale, 0)

            pltpu.async_copy(rows, num_sh.at[src2.at[j]], sem2,
                             add=True).wait()
            dn.wait()
            return 0

        lax.fori_loop(0, qh, chunk, 0)

    half_loop(0)
    half_loop(1)
    plsc.subcore_barrier()

    pltpu.sync_copy(den_sh.at[pl.ds(sid * stripe, stripe)],
                    den_out.at[pl.ds(cid * npad + sid * stripe, stripe)])
    pltpu.sync_copy(num_sh.at[pl.ds(sid * stripe, stripe)],
                    num_out.at[pl.ds(cid * npad + sid * stripe, stripe)])


def _k3(src_full, dst_full, T, h, exu, n, npad, d, ept0, ept1):
    eptm = max(ept0, ept1)
    body = functools.partial(_k3_body, n, npad, d, ept0, ept1)
    return pl.kernel(
        body,
        out_type=[
            jax.ShapeDtypeStruct((NC * npad,), jnp.float32),
            jax.ShapeDtypeStruct((NC * npad, d), jnp.float32),
        ],
        mesh=plsc.VectorSubcoreMesh(
            core_axis_name="c", subcore_axis_name="s",
            num_cores=NC, num_subcores=NS),
        compiler_params=pltpu.CompilerParams(needs_layout_passes=False),
        scratch_types=[
            pltpu.VMEM((eptm // 2,), jnp.int32),
            pltpu.VMEM((eptm // 2,), jnp.float32),
            pltpu.VMEM((eptm // 2 // CH3, CH3), jnp.int32),
            pltpu.VMEM((CH3,), jnp.int32),
            pltpu.VMEM((CH3, 128), jnp.float32),
            pltpu.VMEM((CH3,), jnp.int32),
            pltpu.VMEM((CH3,), jnp.float32),
            pltpu.VMEM_SHARED((npad, 128), jnp.float32),
            pltpu.VMEM_SHARED((npad,), jnp.float32),
            pltpu.SemaphoreType.DMA,
            pltpu.SemaphoreType.DMA,
            pltpu.SemaphoreType.DMA,
        ],
    )(src_full, dst_full, T, h, exu)


def _k4_body(np_ref, dp_ref, out_ref):
    num = np_ref[0] + np_ref[1]
    den = dp_ref[0] + dp_ref[1]
    v = num / den[:, None]
    out_ref[...] = jnp.where(v > 0, v, jnp.exp(v) - 1.0)


def _k4(numerP, denomP, npad, d):
    return pl.pallas_call(
        _k4_body,
        out_shape=jax.ShapeDtypeStruct((npad, d), jnp.float32),
    )(numerP, denomP)


def kernel(x, edge_index, W, b, Wa, ba):
    n, din = x.shape
    d = W.shape[1]
    e = edge_index.shape[1]

    tot = e + n
    avg = -(-tot // (NT * 384)) * 384
    ept0 = int(round(FRAC0 * 2 * avg / 384)) * 384
    ept1 = 2 * avg - ept0
    totp = NS * (ept0 + ept1)
    pad2 = abs(ept0 - ept1)
    pad = totp - tot + pad2
    npad = -(-n // 2048) * 2048

    loop = jnp.arange(n, dtype=jnp.int32)
    zpad = jnp.zeros((pad,), jnp.int32)
    src_full = jnp.concatenate([edge_index[0].astype(jnp.int32), loop, zpad])
    dst_full = jnp.concatenate([edge_index[1].astype(jnp.int32), loop, zpad])

    params = jnp.zeros((8, din), jnp.float32)
    params = params.at[0].set(b)
    params = params.at[1].set(Wa[:d, 0])
    params = params.at[2].set(Wa[d:, 0])
    params = params.at[3, 0].set(ba[0])

    h, a1, a2 = _k1(x, W, params, n, din)
    T, exu = _k2(src_full, dst_full, a1, a2, n, ept0, ept1, totp + pad2)
    denomP, numerP = _k3(src_full, dst_full, T, h, exu,
                         n, npad, d, ept0, ept1)
    out = _k4(numerP.reshape(NC, npad, d), denomP.reshape(NC, npad),
              npad, d)
    return out[:n]


# speedup vs baseline: 1.0092x; 1.0092x over previous
import functools
import jax
import jax.numpy as jnp
from jax import lax
from jax.experimental import pallas as pl
from jax.experimental.pallas import tpu as pltpu, tpu_sc as plsc

NC = 2
NS = 16
NT = NC * NS
CH = 128
CH3 = 96
FRAC0 = 0.587
L = 16


def _k1_body(x_ref, w_ref, p_ref, h_ref, a1_ref, a2_ref):
    h = jnp.dot(x_ref[...], w_ref[...], preferred_element_type=jnp.float32)
    h = h + p_ref[0][None, :]
    h_ref[...] = h
    a1_ref[...] = jnp.sum(h * p_ref[1][None, :], axis=1) + p_ref[3, 0]
    a2_ref[...] = jnp.sum(h * p_ref[2][None, :], axis=1)


def _k1(x, W, params, n, d):
    return pl.pallas_call(
        _k1_body,
        out_shape=[
            jax.ShapeDtypeStruct((n, d), jnp.float32),
            jax.ShapeDtypeStruct((n,), jnp.float32),
            jax.ShapeDtypeStruct((n,), jnp.float32),
        ],
    )(x, W, params)


def _k2_body(n, ept0, ept1, src_hbm, dst_hbm, t_out, s_v, d_v, idx2, val2,
             sem):
    eptm = max(ept0, ept1)
    cid = lax.axis_index("c")
    sid = lax.axis_index("s")
    is0 = cid == 0
    base = jnp.where(is0, sid * ept0, NS * ept0 + sid * ept1)
    qc = jnp.where(is0, ept0 // CH, ept1 // CH)
    pltpu.sync_copy(src_hbm.at[pl.ds(base, eptm)], s_v)
    pltpu.sync_copy(dst_hbm.at[pl.ds(base, eptm)], d_v)

    def build(j, _):
        for c in range(CH // L):
            off = j * CH + c * L
            sv = s_v[pl.ds(off, L)]
            dv = d_v[pl.ds(off, L)]
            idx2[j, pl.ds(c * L, L)] = sv * n + dv
            val2[j, pl.ds(c * L, L)] = (
                base + off + lax.iota(jnp.int32, L))
        return 0

    lax.fori_loop(0, qc, build, 0)

    def scat(j, _):
        pltpu.async_copy(val2.at[j], t_out.at[idx2.at[j]], sem).wait()
        return 0

    lax.fori_loop(0, qc, scat, 0)


def _k2(src_full, dst_full, n, ept0, ept1):
    nn = n * n
    body = functools.partial(_k2_body, n, ept0, ept1)
    return pl.kernel(
        body,
        out_type=jax.ShapeDtypeStruct((nn,), jnp.int32),
        mesh=plsc.VectorSubcoreMesh(
            core_axis_name="c", subcore_axis_name="s",
            num_cores=NC, num_subcores=NS),
        compiler_params=pltpu.CompilerParams(needs_layout_passes=False),
        scratch_types=[
            pltpu.VMEM((max(ept0, ept1),), jnp.int32),
            pltpu.VMEM((max(ept0, ept1),), jnp.int32),
            pltpu.VMEM((max(ept0, ept1) // CH, CH), jnp.int32),
            pltpu.VMEM((max(ept0, ept1) // CH, CH), jnp.int32),
            pltpu.SemaphoreType.DMA,
        ],
    )(src_full, dst_full)


def _k3_body(n, npad, d, ept0, ept1,
             src_hbm, dst_hbm, t_hbm, h_hbm, a1_hbm, a2_hbm,
             den_out, num_out,
             d_v, src2, eid_b, a1g_b, a2g_b, rows, w_v, ex_v,
             num_sh, den_sh, sem, sem2, sem3):
    cid = lax.axis_index("c")
    sid = lax.axis_index("s")
    is0 = cid == 0
    base = jnp.where(is0, sid * ept0, NS * ept0 + sid * ept1)
    qc = jnp.where(is0, ept0 // CH3, ept1 // CH3)
    stripe = npad // NS

    pltpu.sync_copy(dst_hbm.at[pl.ds(base, max(ept0, ept1))], d_v)

    zero16 = jnp.zeros((L,), jnp.float32)

    def zrows(i, _):
        for c in range(d // L):
            rows[i, pl.ds(c * L, L)] = zero16
        return 0
    lax.fori_loop(0, CH3, zrows, 0)

    zoffs = sorted({min(t * CH3, stripe - CH3)
                    for t in range(-(-stripe // CH3))})
    for t in zoffs:
        pltpu.sync_copy(rows, num_sh.at[pl.ds(sid * stripe + t, CH3)])
    doffs = sorted({min(t * d, stripe - d) for t in range(-(-stripe // d))})
    for t in doffs:
        pltpu.sync_copy(rows.at[0],
                        den_sh.at[pl.ds(sid * stripe + t, d)])
    plsc.subcore_barrier()

    def chunk(j, _):
        pltpu.sync_copy(src_hbm.at[pl.ds(base + j * CH3, CH3)], src2.at[j])
        for c in range(CH3 // L):
            off = j * CH3 + c * L
            eid_b[pl.ds(c * L, L)] = (
                src2[j, pl.ds(c * L, L)] * n + d_v[pl.ds(off, L)])
        g1 = pltpu.async_copy(t_hbm.at[eid_b], w_v, sem)
        g2 = pltpu.async_copy(a1_hbm.at[src2.at[j]], a1g_b, sem)
        g3 = pltpu.async_copy(
            a2_hbm.at[d_v.at[pl.ds(j * CH3, CH3)]], a2g_b, sem)
        cp = pltpu.async_copy(h_hbm.at[d_v.at[pl.ds(j * CH3, CH3)]], rows,
                              sem2)
        g1.wait(); g2.wait(); g3.wait()
        for c in range(CH3 // L):
            off = j * CH3 + c * L
            k = base + off + lax.iota(jnp.int32, L)
            e = a1g_b[pl.ds(c * L, L)] + a2g_b[pl.ds(c * L, L)]
            e = jnp.where(e > 0, e, e * jnp.float32(0.01))
            ex = jnp.where(w_v[pl.ds(c * L, L)] == k,
                           jnp.exp(e), jnp.float32(0.0))
            ex_v[pl.ds(c * L, L)] = ex
        dn = pltpu.async_copy(ex_v, den_sh.at[src2.at[j]], sem3, add=True)
        cp.wait()

        def scale(g, _):
            exg = ex_v[pl.ds(g * L, L)]
            for lane in range(L):
                i = g * L + lane
                s = jnp.full((L,), exg[lane], jnp.float32)
                for c in range(d // L):
                    rows[i, pl.ds(c * L, L)] = rows[i, pl.ds(c * L, L)] * s
            return 0
        lax.fori_loop(0, CH3 // L, scale, 0)

        pltpu.async_copy(rows, num_sh.at[src2.at[j]], sem2, add=True).wait()
        dn.wait()
        return 0

    lax.fori_loop(0, qc, chunk, 0)
    plsc.subcore_barrier()

    pltpu.sync_copy(den_sh.at[pl.ds(sid * stripe, stripe)],
                    den_out.at[pl.ds(cid * npad + sid * stripe, stripe)])
    pltpu.sync_copy(num_sh.at[pl.ds(sid * stripe, stripe)],
                    num_out.at[pl.ds(cid * npad + sid * stripe, stripe)])


def _k3(src_full, dst_full, T, h, a1, a2, n, npad, d, ept0, ept1):
    body = functools.partial(_k3_body, n, npad, d, ept0, ept1)
    return pl.kernel(
        body,
        out_type=[
            jax.ShapeDtypeStruct((NC * npad,), jnp.float32),
            jax.ShapeDtypeStruct((NC * npad, d), jnp.float32),
        ],
        mesh=plsc.VectorSubcoreMesh(
            core_axis_name="c", subcore_axis_name="s",
            num_cores=NC, num_subcores=NS),
        compiler_params=pltpu.CompilerParams(needs_layout_passes=False),
        scratch_types=[
            pltpu.VMEM((max(ept0, ept1),), jnp.int32),
            pltpu.VMEM((max(ept0, ept1) // CH3, CH3), jnp.int32),
            pltpu.VMEM((CH3,), jnp.int32),
            pltpu.VMEM((CH3,), jnp.float32),
            pltpu.VMEM((CH3,), jnp.float32),
            pltpu.VMEM((CH3, 128), jnp.float32),
            pltpu.VMEM((CH3,), jnp.int32),
            pltpu.VMEM((CH3,), jnp.float32),
            pltpu.VMEM_SHARED((npad, 128), jnp.float32),
            pltpu.VMEM_SHARED((npad,), jnp.float32),
            pltpu.SemaphoreType.DMA,
            pltpu.SemaphoreType.DMA,
            pltpu.SemaphoreType.DMA,
        ],
    )(src_full, dst_full, T, h, a1, a2)


def _k4_body(np_ref, dp_ref, out_ref):
    num = np_ref[0] + np_ref[1]
    den = dp_ref[0] + dp_ref[1]
    v = num / den[:, None]
    out_ref[...] = jnp.where(v > 0, v, jnp.exp(v) - 1.0)


def _k4(numerP, denomP, npad, d):
    return pl.pallas_call(
        _k4_body,
        out_shape=jax.ShapeDtypeStruct((npad, d), jnp.float32),
    )(numerP, denomP)


def kernel(x, edge_index, W, b, Wa, ba):
    n, din = x.shape
    d = W.shape[1]
    e = edge_index.shape[1]

    tot = e + n
    avg = -(-tot // (NT * 384)) * 384
    ept0 = int(round(FRAC0 * 2 * avg / 384)) * 384
    ept1 = 2 * avg - ept0
    totp = NS * (ept0 + ept1)
    pad = totp - tot + abs(ept0 - ept1)
    npad = -(-n // 2048) * 2048

    loop = jnp.arange(n, dtype=jnp.int32)
    zpad = jnp.zeros((pad,), jnp.int32)
    src_full = jnp.concatenate([edge_index[0].astype(jnp.int32), loop, zpad])
    dst_full = jnp.concatenate([edge_index[1].astype(jnp.int32), loop, zpad])

    params = jnp.zeros((8, din), jnp.float32)
    params = params.at[0].set(b)
    params = params.at[1].set(Wa[:d, 0])
    params = params.at[2].set(Wa[d:, 0])
    params = params.at[3, 0].set(ba[0])

    h, a1, a2 = _k1(x, W, params, n, din)
    T = _k2(src_full, dst_full, n, ept0, ept1)
    denomP, numerP = _k3(src_full, dst_full, T, h, a1, a2,
                         n, npad, d, ept0, ept1)
    out = _k4(numerP.reshape(NC, npad, d), denomP.reshape(NC, npad),
              npad, d)
    return out[:n]
